# baseline (speedup 1.0000x reference)
import jax
import jax.numpy as jnp
from jax import lax
from jax.experimental import pallas as pl
from jax.experimental.pallas import tpu as pltpu
from jax.experimental.pallas import tpu_sc as plsc

USER_NUM = 1000000
ITEM_NUM = 100000
D = 32
B = 4096
NEG = 50
NBR = 10
W1, W2, W3, W4 = 1e-07, 1.0, 1e-07, 1.0
NEGATIVE_WEIGHT = 10.0
GAMMA = 1e-4
LAMBDA = 2.75

NW = 32
BW = B // NW
NEGW = B * NEG // NW
NBRW = B * NBR // NW
NEG_CHUNK = 1600


def _sc_gather_body(users_h, pos_h, negf_h, nbre_h,
                    ueflat_h, iemb_h, bu_h, bi_h, iin_h, iic_h,
                    ut_out, p_out, n_out, nbr_out,
                    bu_out, bp_out, bn_out, sim_out,
                    idx_u, idx_p, idx_n, idx_e, nbr_i, idx32, u32buf,
                    rows, vals, sem):
    wid = lax.axis_index("s") * 2 + lax.axis_index("c")
    ubase = wid * BW
    negbase = wid * NEGW
    nbase = wid * NBRW

    pltpu.sync_copy(users_h.at[pl.ds(ubase, BW)], idx_u)
    pltpu.sync_copy(pos_h.at[pl.ds(ubase, BW)], idx_p)
    pltpu.sync_copy(negf_h.at[pl.ds(negbase, NEGW)], idx_n)
    pltpu.sync_copy(nbre_h.at[pl.ds(nbase, NBRW)], idx_e)

    pltpu.sync_copy(bu_h.at[idx_u], vals.at[pl.ds(0, BW)])
    pltpu.sync_copy(vals.at[pl.ds(0, BW)], bu_out.at[pl.ds(ubase, BW)])

    for d in range(32):
        for c in range(0, BW, 16):
            idx32[d, pl.ds(c, 16)] = idx_u[pl.ds(c, 16)] + (d * USER_NUM)
    plane_cps = [
        pltpu.async_copy(ueflat_h.at[idx32.at[d]], u32buf.at[d], sem)
        for d in range(32)
    ]
    for cp in plane_cps:
        cp.wait()
    pltpu.sync_copy(u32buf, ut_out.at[:, pl.ds(ubase, BW)])

    pltpu.sync_copy(iemb_h.at[idx_p], rows.at[pl.ds(0, BW)])
    pltpu.sync_copy(rows.at[pl.ds(0, BW)], p_out.at[pl.ds(ubase, BW)])
    pltpu.sync_copy(bi_h.at[idx_p], vals.at[pl.ds(0, BW)])
    pltpu.sync_copy(vals.at[pl.ds(0, BW)], bp_out.at[pl.ds(ubase, BW)])

    @pl.loop(0, NEGW, step=NEG_CHUNK)
    def _(c):
        pltpu.sync_copy(iemb_h.at[idx_n.at[pl.ds(c, NEG_CHUNK)]], rows)
        pltpu.sync_copy(rows, n_out.at[pl.ds(negbase + c, NEG_CHUNK)])
        pltpu.sync_copy(bi_h.at[idx_n.at[pl.ds(c, NEG_CHUNK)]], vals)
        pltpu.sync_copy(vals, bn_out.at[pl.ds(negbase + c, NEG_CHUNK)])

    pltpu.sync_copy(iin_h.at[idx_e], nbr_i)
    pltpu.sync_copy(iic_h.at[idx_e], vals.at[pl.ds(0, NBRW)])
    pltpu.sync_copy(vals.at[pl.ds(0, NBRW)], sim_out.at[pl.ds(nbase, NBRW)])
    pltpu.sync_copy(iemb_h.at[nbr_i], rows.at[pl.ds(0, NBRW)])
    pltpu.sync_copy(rows.at[pl.ds(0, NBRW)], nbr_out.at[pl.ds(nbase, NBRW)])


def _sc_gather(users, pos_items, neg_flat, nbr_elem,
               ue_flat, item_embeds, beta_uD, beta_iD,
               iin_flat, iic_flat):
    f32 = jnp.float32
    mesh = plsc.VectorSubcoreMesh(core_axis_name="c", subcore_axis_name="s")
    out_type = (
        jax.ShapeDtypeStruct((D, B), f32),
        jax.ShapeDtypeStruct((B, D), f32),
        jax.ShapeDtypeStruct((B * NEG, D), f32),
        jax.ShapeDtypeStruct((B * NBR, D), f32),
        jax.ShapeDtypeStruct((B,), f32),
        jax.ShapeDtypeStruct((B,), f32),
        jax.ShapeDtypeStruct((B * NEG,), f32),
        jax.ShapeDtypeStruct((B * NBR,), f32),
    )
    scratch = [
        pltpu.VMEM((BW,), jnp.int32),
        pltpu.VMEM((BW,), jnp.int32),
        pltpu.VMEM((NEGW,), jnp.int32),
        pltpu.VMEM((NBRW,), jnp.int32),
        pltpu.VMEM((NBRW,), jnp.int32),
        pltpu.VMEM((D, BW), jnp.int32),
        pltpu.VMEM((D, BW), f32),
        pltpu.VMEM((NEG_CHUNK, D), f32),
        pltpu.VMEM((NEG_CHUNK,), f32),
        pltpu.SemaphoreType.DMA,
    ]
    k = pl.kernel(_sc_gather_body, out_type=out_type, mesh=mesh,
                  scratch_types=scratch,
                  compiler_params=pltpu.CompilerParams(
                      use_tc_tiling_on_sc=False))
    return k(users, pos_items, neg_flat, nbr_elem,
             ue_flat, item_embeds, beta_uD, beta_iD, iin_flat, iic_flat)


def _norm_body(u_ref, i_ref, o_ref, acc):
    step = pl.program_id(0)

    @pl.when(step == 0)
    def _():
        acc[0] = 0.0

    part = jnp.sum(u_ref[...] * u_ref[...]) + jnp.sum(i_ref[...] * i_ref[...])
    acc[0] += part

    @pl.when(step == pl.num_programs(0) - 1)
    def _():
        o_ref[0] = acc[0] * 0.5


def _norm(ue_flat, it_flat):
    ue = ue_flat.reshape(250000, 128)
    ie = it_flat.reshape(25000, 128)
    grid = 25
    out = pl.pallas_call(
        _norm_body,
        grid=(grid,),
        in_specs=[
            pl.BlockSpec((10000, 128), lambda i: (i, 0)),
            pl.BlockSpec((1000, 128), lambda i: (i, 0)),
        ],
        out_specs=pl.BlockSpec(memory_space=pltpu.SMEM),
        out_shape=jax.ShapeDtypeStruct((1,), jnp.float32),
        scratch_shapes=[pltpu.SMEM((1,), jnp.float32)],
    )(ue, ie)
    return out[0]


def _loss_body(ut_ref, p_ref, bu_ref, bp_ref, n_ref, bn_ref, nbr_ref, sim_ref,
               o_ref, acc):
    step = pl.program_id(0)

    @pl.when(step == 0)
    def _():
        acc[0] = 0.0

    u = jnp.transpose(ut_ref[...])
    p = p_ref[...]
    bu = bu_ref[...]
    bp = bp_ref[...]
    n = n_ref[...]
    bn = bn_ref[...]
    nbr = nbr_ref[...]
    sim = sim_ref[...]

    pos_scores = jnp.sum(u * p, axis=-1, keepdims=True)
    pos_w = W1 + W2 * (bu * bp)
    pos_part = jnp.sum(pos_w * jax.nn.softplus(-pos_scores))

    neg_scores = jnp.sum(u[:, None, :] * n, axis=-1)
    neg_w = W3 + W4 * (bu * bn)
    neg_part = jnp.sum(neg_w * jax.nn.softplus(neg_scores))

    nbr_scores = jnp.sum(u[:, None, :] * nbr, axis=-1)
    i_part = jnp.sum(sim * jax.nn.softplus(-nbr_scores))

    acc[0] += (pos_part + (NEGATIVE_WEIGHT / NEG) * neg_part + LAMBDA * i_part)

    @pl.when(step == pl.num_programs(0) - 1)
    def _():
        o_ref[0] = acc[0]


def _loss(ut, p, bu, bp, n, bn, nbr, sim):
    grid = 16
    bc = B // grid
    n3 = n.reshape(B, NEG, D)
    nbr3 = nbr.reshape(B, NBR, D)
    bu2 = bu.reshape(B, 1)
    bp2 = bp.reshape(B, 1)
    bn2 = bn.reshape(B, NEG)
    sim2 = sim.reshape(B, NBR)
    out = pl.pallas_call(
        _loss_body,
        grid=(grid,),
        in_specs=[
            pl.BlockSpec((D, bc), lambda i: (0, i)),
            pl.BlockSpec((bc, D), lambda i: (i, 0)),
            pl.BlockSpec((bc, 1), lambda i: (i, 0)),
            pl.BlockSpec((bc, 1), lambda i: (i, 0)),
            pl.BlockSpec((bc, NEG, D), lambda i: (i, 0, 0)),
            pl.BlockSpec((bc, NEG), lambda i: (i, 0)),
            pl.BlockSpec((bc, NBR, D), lambda i: (i, 0, 0)),
            pl.BlockSpec((bc, NBR), lambda i: (i, 0)),
        ],
        out_specs=pl.BlockSpec(memory_space=pltpu.SMEM),
        out_shape=jax.ShapeDtypeStruct((1,), jnp.float32),
        scratch_shapes=[pltpu.SMEM((1,), jnp.float32)],
    )(ut, p, bu2, bp2, n3, bn2, nbr3, sim2)
    return out[0]


@jax.jit
def _run(users, pos_items, neg_items, user_embeds, item_embeds,
         beta_uD, beta_iD, ii_neighbor_mat, ii_constraint_mat):
    i32 = jnp.int32
    users = users.astype(i32)
    pos_items = pos_items.astype(i32)
    neg_flat = neg_items.astype(i32).reshape(-1)
    nbr_elem = (pos_items * NBR)[:, None] + jnp.arange(NBR, dtype=i32)
    nbr_elem = nbr_elem.reshape(-1)
    iin_flat = ii_neighbor_mat.astype(i32).reshape(-1)
    iic_flat = ii_constraint_mat.reshape(-1)

    ue_flat = jnp.transpose(user_embeds).reshape(-1)
    it_flat = jnp.transpose(item_embeds).reshape(-1)

    ut, p, n, nbr, bu, bp, bn, sim = _sc_gather(
        users, pos_items, neg_flat, nbr_elem,
        ue_flat, item_embeds, beta_uD, beta_iD, iin_flat, iic_flat)

    norm = _norm(ue_flat, it_flat)
    loss = _loss(ut, p, bu, bp, n, bn, nbr, sim)
    return loss + GAMMA * norm


def kernel(users, pos_items, neg_items, user_embeds, item_embeds,
           beta_uD, beta_iD, ii_neighbor_mat, ii_constraint_mat):
    return _run(users, pos_items, neg_items, user_embeds, item_embeds,
                beta_uD, beta_iD, ii_neighbor_mat, ii_constraint_mat)


# speedup vs baseline: 7.2491x; 7.2491x over previous
import jax
import jax.numpy as jnp
from jax import lax
from jax.experimental import pallas as pl
from jax.experimental.pallas import tpu as pltpu
from jax.experimental.pallas import tpu_sc as plsc

USER_NUM = 1000000
ITEM_NUM = 100000
D = 32
B = 4096
NEG = 50
NBR = 10
W1, W2, W3, W4 = 1e-07, 1.0, 1e-07, 1.0
NEGATIVE_WEIGHT = 10.0
GAMMA = 1e-4
LAMBDA = 2.75

NW = 32
BW = B // NW
NEGW = B * NEG // NW
NBRW = B * NBR // NW
NEG_CHUNK = 1600


def _sc_gather_body(users_h, pos_h, negf_h, nbre_h,
                    uemb_h, iemb_h, bu_h, bi_h, iin_h, iic_h,
                    u_out, p_out, n_out, nbr_out,
                    bu_out, bp_out, bn_out, sim_out,
                    idx_u, idx_p, idx_n, idx_e, nbr_i, rows, vals):
    wid = lax.axis_index("s") * 2 + lax.axis_index("c")
    ubase = wid * BW
    negbase = wid * NEGW
    nbase = wid * NBRW

    pltpu.sync_copy(users_h.at[pl.ds(ubase, BW)], idx_u)
    pltpu.sync_copy(pos_h.at[pl.ds(ubase, BW)], idx_p)
    pltpu.sync_copy(negf_h.at[pl.ds(negbase, NEGW)], idx_n)
    pltpu.sync_copy(nbre_h.at[pl.ds(nbase, NBRW)], idx_e)

    pltpu.sync_copy(uemb_h.at[idx_u], rows.at[pl.ds(0, BW)])
    pltpu.sync_copy(rows.at[pl.ds(0, BW)], u_out.at[pl.ds(ubase, BW)])
    pltpu.sync_copy(bu_h.at[idx_u], vals.at[pl.ds(0, BW)])
    pltpu.sync_copy(vals.at[pl.ds(0, BW)], bu_out.at[pl.ds(ubase, BW)])

    pltpu.sync_copy(iemb_h.at[idx_p], rows.at[pl.ds(0, BW)])
    pltpu.sync_copy(rows.at[pl.ds(0, BW)], p_out.at[pl.ds(ubase, BW)])
    pltpu.sync_copy(bi_h.at[idx_p], vals.at[pl.ds(0, BW)])
    pltpu.sync_copy(vals.at[pl.ds(0, BW)], bp_out.at[pl.ds(ubase, BW)])

    @pl.loop(0, NEGW, step=NEG_CHUNK)
    def _(c):
        pltpu.sync_copy(iemb_h.at[idx_n.at[pl.ds(c, NEG_CHUNK)]], rows)
        pltpu.sync_copy(rows, n_out.at[pl.ds(negbase + c, NEG_CHUNK)])
        pltpu.sync_copy(bi_h.at[idx_n.at[pl.ds(c, NEG_CHUNK)]], vals)
        pltpu.sync_copy(vals, bn_out.at[pl.ds(negbase + c, NEG_CHUNK)])

    pltpu.sync_copy(iin_h.at[idx_e], nbr_i)
    pltpu.sync_copy(iic_h.at[idx_e], vals.at[pl.ds(0, NBRW)])
    pltpu.sync_copy(vals.at[pl.ds(0, NBRW)], sim_out.at[pl.ds(nbase, NBRW)])
    pltpu.sync_copy(iemb_h.at[nbr_i], rows.at[pl.ds(0, NBRW)])
    pltpu.sync_copy(rows.at[pl.ds(0, NBRW)], nbr_out.at[pl.ds(nbase, NBRW)])


def _sc_gather(users, pos_items, neg_flat, nbr_elem,
               user_embeds, item_embeds, beta_uD, beta_iD,
               iin_flat, iic_flat):
    f32 = jnp.float32
    mesh = plsc.VectorSubcoreMesh(core_axis_name="c", subcore_axis_name="s")
    out_type = (
        jax.ShapeDtypeStruct((B, D), f32),
        jax.ShapeDtypeStruct((B, D), f32),
        jax.ShapeDtypeStruct((B * NEG, D), f32),
        jax.ShapeDtypeStruct((B * NBR, D), f32),
        jax.ShapeDtypeStruct((B,), f32),
        jax.ShapeDtypeStruct((B,), f32),
        jax.ShapeDtypeStruct((B * NEG,), f32),
        jax.ShapeDtypeStruct((B * NBR,), f32),
    )
    scratch = [
        pltpu.VMEM((BW,), jnp.int32),
        pltpu.VMEM((BW,), jnp.int32),
        pltpu.VMEM((NEGW,), jnp.int32),
        pltpu.VMEM((NBRW,), jnp.int32),
        pltpu.VMEM((NBRW,), jnp.int32),
        pltpu.VMEM((NEG_CHUNK, D), f32),
        pltpu.VMEM((NEG_CHUNK,), f32),
    ]
    k = pl.kernel(_sc_gather_body, out_type=out_type, mesh=mesh,
                  scratch_types=scratch,
                  compiler_params=pltpu.CompilerParams(
                      use_tc_tiling_on_sc=False))
    return k(users, pos_items, neg_flat, nbr_elem,
             user_embeds, item_embeds, beta_uD, beta_iD, iin_flat, iic_flat)


U_CHUNK = 65536
U_TAIL = USER_NUM - 15 * U_CHUNK
I_CHUNK = 12800
I_TAIL = ITEM_NUM - 7 * I_CHUNK


def _norm_jobs():
    jobs = []
    k = 0
    for g in range(4):
        for c in range(15):
            jobs.append((True, g, c * U_CHUNK, U_CHUNK, k % 2))
            k += 1
        jobs.append((True, g, 15 * U_CHUNK, U_TAIL, 2))
    for g in range(4):
        for c in range(7):
            jobs.append((False, g, c * I_CHUNK, I_CHUNK, k % 2))
            k += 1
        jobs.append((False, g, 7 * I_CHUNK, I_TAIL, 3))
    return jobs


def _norm_body(u_hbm, i_hbm, o_ref, buf0, buf1, buf2, buf3,
               sem0, sem1, sem2, sem3, acc):
    jobs = _norm_jobs()
    bufs = (buf0, buf1, buf2, buf3)
    sems = (sem0, sem1, sem2, sem3)

    def start(k):
        is_u, g, off, ln, b = jobs[k]
        src = (u_hbm if is_u else i_hbm).at[pl.ds(g * 8, 8), pl.ds(off, ln)]
        if ln == bufs[b].shape[1]:
            dst = bufs[b]
        else:
            dst = bufs[b].at[:, pl.ds(0, ln)]
        cp = pltpu.make_async_copy(src, dst, sems[b])
        cp.start()
        return cp

    acc[0] = 0.0
    cps = {0: start(0)}
    for k in range(len(jobs)):
        if k + 1 < len(jobs):
            cps[k + 1] = start(k + 1)
        cps[k].wait()
        ln, b = jobs[k][3], jobs[k][4]
        if ln == bufs[b].shape[1]:
            x = bufs[b][...]
        else:
            x = bufs[b][:, pl.ds(0, ln)]
        acc[0] += jnp.sum(x * x)
    o_ref[0] = acc[0] * 0.5


def _norm(ue_t, it_t):
    out = pl.pallas_call(
        _norm_body,
        in_specs=[
            pl.BlockSpec(memory_space=pl.ANY),
            pl.BlockSpec(memory_space=pl.ANY),
        ],
        out_specs=pl.BlockSpec(memory_space=pltpu.SMEM),
        out_shape=jax.ShapeDtypeStruct((1,), jnp.float32),
        scratch_shapes=[
            pltpu.VMEM((8, U_CHUNK), jnp.float32),
            pltpu.VMEM((8, U_CHUNK), jnp.float32),
            pltpu.VMEM((8, U_TAIL), jnp.float32),
            pltpu.VMEM((8, I_TAIL), jnp.float32),
            pltpu.SemaphoreType.DMA,
            pltpu.SemaphoreType.DMA,
            pltpu.SemaphoreType.DMA,
            pltpu.SemaphoreType.DMA,
            pltpu.SMEM((1,), jnp.float32),
        ],
    )(ue_t, it_t)
    return out[0]


def _loss_body(u_ref, p_ref, bu_ref, bp_ref, n_ref, bn_ref, nbr_ref, sim_ref,
               o_ref, acc):
    step = pl.program_id(0)

    @pl.when(step == 0)
    def _():
        acc[0] = 0.0

    u = u_ref[...]
    p = p_ref[...]
    bu = bu_ref[...]
    bp = bp_ref[...]
    n = n_ref[...]
    bn = bn_ref[...]
    nbr = nbr_ref[...]
    sim = sim_ref[...]

    pos_scores = jnp.sum(u * p, axis=-1, keepdims=True)
    pos_w = W1 + W2 * (bu * bp)
    pos_part = jnp.sum(pos_w * jax.nn.softplus(-pos_scores))

    neg_scores = jnp.sum(u[:, None, :] * n, axis=-1)
    neg_w = W3 + W4 * (bu * bn)
    neg_part = jnp.sum(neg_w * jax.nn.softplus(neg_scores))

    nbr_scores = jnp.sum(u[:, None, :] * nbr, axis=-1)
    i_part = jnp.sum(sim * jax.nn.softplus(-nbr_scores))

    acc[0] += (pos_part + (NEGATIVE_WEIGHT / NEG) * neg_part + LAMBDA * i_part)

    @pl.when(step == pl.num_programs(0) - 1)
    def _():
        o_ref[0] = acc[0]


def _loss(u, p, bu, bp, n, bn, nbr, sim):
    grid = 16
    bc = B // grid
    n3 = n.reshape(B, NEG, D)
    nbr3 = nbr.reshape(B, NBR, D)
    bu2 = bu.reshape(B, 1)
    bp2 = bp.reshape(B, 1)
    bn2 = bn.reshape(B, NEG)
    sim2 = sim.reshape(B, NBR)
    out = pl.pallas_call(
        _loss_body,
        grid=(grid,),
        in_specs=[
            pl.BlockSpec((bc, D), lambda i: (i, 0)),
            pl.BlockSpec((bc, D), lambda i: (i, 0)),
            pl.BlockSpec((bc, 1), lambda i: (i, 0)),
            pl.BlockSpec((bc, 1), lambda i: (i, 0)),
            pl.BlockSpec((bc, NEG, D), lambda i: (i, 0, 0)),
            pl.BlockSpec((bc, NEG), lambda i: (i, 0)),
            pl.BlockSpec((bc, NBR, D), lambda i: (i, 0, 0)),
            pl.BlockSpec((bc, NBR), lambda i: (i, 0)),
        ],
        out_specs=pl.BlockSpec(memory_space=pltpu.SMEM),
        out_shape=jax.ShapeDtypeStruct((1,), jnp.float32),
        scratch_shapes=[pltpu.SMEM((1,), jnp.float32)],
    )(u, p, bu2, bp2, n3, bn2, nbr3, sim2)
    return out[0]


@jax.jit
def _run(users, pos_items, neg_items, user_embeds, item_embeds,
         beta_uD, beta_iD, ii_neighbor_mat, ii_constraint_mat):
    i32 = jnp.int32
    users = users.astype(i32)
    pos_items = pos_items.astype(i32)
    neg_flat = neg_items.astype(i32).reshape(-1)
    nbr_elem = (pos_items * NBR)[:, None] + jnp.arange(NBR, dtype=i32)
    nbr_elem = nbr_elem.reshape(-1)
    iin_flat = ii_neighbor_mat.astype(i32).reshape(-1)
    iic_flat = ii_constraint_mat.reshape(-1)

    u, p, n, nbr, bu, bp, bn, sim = _sc_gather(
        users, pos_items, neg_flat, nbr_elem,
        user_embeds, item_embeds, beta_uD, beta_iD, iin_flat, iic_flat)

    norm = _norm(jnp.transpose(user_embeds), jnp.transpose(item_embeds))
    loss = _loss(u, p, bu, bp, n, bn, nbr, sim)
    return loss + GAMMA * norm


def kernel(users, pos_items, neg_items, user_embeds, item_embeds,
           beta_uD, beta_iD, ii_neighbor_mat, ii_constraint_mat):
    return _run(users, pos_items, neg_items, user_embeds, item_embeds,
                beta_uD, beta_iD, ii_neighbor_mat, ii_constraint_mat)
